# Initial kernel scaffold; baseline (speedup 1.0000x reference)
import jax
import jax.numpy as jnp
from jax.experimental import pallas as pl


def kernel(x, edge_index, W_self0, W_neigh0, b0, W_self1, W_neigh1, b1, Wr0, br0, Wr1, br1):
    raise NotImplementedError("write your pallas kernel here")


# speedup vs baseline: 4.3254x; 4.3254x over previous
import functools

import jax
import jax.numpy as jnp
from jax import lax
from jax.experimental import pallas as pl
from jax.experimental.pallas import tpu as pltpu
from jax.experimental.pallas import tpu_sc as plsc

N = 10000
E = 320000
D = 128
NC = 2
NS = 16
EPC = E // NC
EPW = EPC // NS
K = 80
NP = 10240
RPW = NP // NS


@functools.cache
def _sc_mesh():
  return plsc.VectorSubcoreMesh(core_axis_name="c", subcore_axis_name="s")


def _sc_agg(h, src, dst, zeros_rows):
  @functools.partial(
      pl.kernel,
      mesh=_sc_mesh(),
      out_type=jax.ShapeDtypeStruct((NC * NP, D), jnp.float32),
      scratch_types=[
          pltpu.VMEM((K,), jnp.int32),
          pltpu.VMEM((K,), jnp.int32),
          pltpu.VMEM((K, D), jnp.float32),
          pltpu.VMEM_SHARED((NP, D), jnp.float32),
          pltpu.SemaphoreType.DMA,
      ],
  )
  def body(h_hbm, src_hbm, dst_hbm, zr_hbm, agg_out, src_v, dst_v, rows_v,
           acc_sh, sem):
    cid = lax.axis_index("c")
    sid = lax.axis_index("s")
    row0 = sid * RPW
    pltpu.sync_copy(zr_hbm, rows_v)

    @pl.loop(0, RPW, step=K)
    def _(j):
      pltpu.sync_copy(rows_v, acc_sh.at[pl.ds(row0 + j, K)])

    plsc.subcore_barrier()
    base = cid * EPC + sid * EPW

    @pl.loop(0, EPW, step=K)
    def _(off):
      b = base + off
      pltpu.sync_copy(src_hbm.at[pl.ds(b, K)], src_v)
      pltpu.async_copy(h_hbm.at[src_v], rows_v, sem).wait()
      pltpu.sync_copy(dst_hbm.at[pl.ds(b, K)], dst_v)
      pltpu.sync_copy(rows_v, acc_sh.at[dst_v], add=True)

    plsc.subcore_barrier()
    out_r = cid * NP + row0

    @pl.loop(0, RPW, step=K)
    def _(j):
      pltpu.sync_copy(acc_sh.at[pl.ds(row0 + j, K)], rows_v)
      pltpu.sync_copy(rows_v, agg_out.at[pl.ds(out_r + j, K)])

  return body(h, src, dst, zeros_rows).reshape(NC, NP, D)


def _sc_deg(dst, zeros_rows, ones_rows):
  @functools.partial(
      pl.kernel,
      mesh=_sc_mesh(),
      out_type=jax.ShapeDtypeStruct((NC * NP, D), jnp.float32),
      scratch_types=[
          pltpu.VMEM((K,), jnp.int32),
          pltpu.VMEM((K, D), jnp.float32),
          pltpu.VMEM_SHARED((NP, D), jnp.float32),
          pltpu.SemaphoreType.DMA,
      ],
  )
  def body(dst_hbm, zr_hbm, ones_hbm, deg_out, dst_v, rows_v, acc_sh, sem):
    cid = lax.axis_index("c")
    sid = lax.axis_index("s")
    row0 = sid * RPW
    pltpu.sync_copy(zr_hbm, rows_v)

    @pl.loop(0, RPW, step=K)
    def _(j):
      pltpu.sync_copy(rows_v, acc_sh.at[pl.ds(row0 + j, K)])

    plsc.subcore_barrier()
    pltpu.sync_copy(ones_hbm, rows_v)
    base = cid * EPC + sid * EPW

    @pl.loop(0, EPW, step=K)
    def _(off):
      pltpu.sync_copy(dst_hbm.at[pl.ds(base + off, K)], dst_v)
      pltpu.sync_copy(rows_v, acc_sh.at[dst_v], add=True)

    plsc.subcore_barrier()
    out_r = cid * NP + row0

    @pl.loop(0, RPW, step=K)
    def _(j):
      pltpu.sync_copy(acc_sh.at[pl.ds(row0 + j, K)], rows_v)
      pltpu.sync_copy(rows_v, deg_out.at[pl.ds(out_r + j, K)])

  return body(dst, zeros_rows, ones_rows).reshape(NC, NP, D)


ROWS_BLK = 1000


def _tc_layer1_body(h_ref, agg_ref, deg_ref, ws_ref, wn_ref, b_ref, out_ref):
  agg = agg_ref[0] + agg_ref[1]
  deg = deg_ref[0] + deg_ref[1]
  dinv = 1.0 / jnp.maximum(deg, 1.0)
  hs = jnp.dot(h_ref[...], ws_ref[...], preferred_element_type=jnp.float32)
  hn = jnp.dot(agg, wn_ref[...], preferred_element_type=jnp.float32)
  out_ref[...] = jnp.maximum(hs + hn * dinv + b_ref[...], 0.0)


def _tc_layer1(h, aggp, degs, W_self, W_neigh, b):
  grid = (N // ROWS_BLK,)
  return pl.pallas_call(
      _tc_layer1_body,
      grid=grid,
      in_specs=[
          pl.BlockSpec((ROWS_BLK, D), lambda i: (i, 0)),
          pl.BlockSpec((NC, ROWS_BLK, D), lambda i: (0, i, 0)),
          pl.BlockSpec((NC, ROWS_BLK, 1), lambda i: (0, i, 0)),
          pl.BlockSpec((D, D), lambda i: (0, 0)),
          pl.BlockSpec((D, D), lambda i: (0, 0)),
          pl.BlockSpec((1, D), lambda i: (0, 0)),
      ],
      out_specs=pl.BlockSpec((ROWS_BLK, D), lambda i: (i, 0)),
      out_shape=jax.ShapeDtypeStruct((N, D), jnp.float32),
  )(h, aggp, degs, W_self, W_neigh, b.reshape(1, D))


def _tc_layer2_body(h_ref, agg_ref, deg_ref, ws_ref, wn_ref, b_ref,
                    wr0_ref, br0_ref, wr1_ref, br1_ref, out_ref):
  agg = agg_ref[0] + agg_ref[1]
  deg = deg_ref[0] + deg_ref[1]
  dinv = 1.0 / jnp.maximum(deg, 1.0)
  hs = jnp.dot(h_ref[...], ws_ref[...], preferred_element_type=jnp.float32)
  hn = jnp.dot(agg, wn_ref[...], preferred_element_type=jnp.float32)
  h2 = jnp.maximum(hs + hn * dinv + b_ref[...], 0.0)
  r = jnp.maximum(
      jnp.dot(h2, wr0_ref[...], preferred_element_type=jnp.float32)
      + br0_ref[...], 0.0)
  out_ref[...] = (
      jnp.dot(r, wr1_ref[...], preferred_element_type=jnp.float32)
      + br1_ref[...])


def _tc_layer2(h, aggp, degs, W_self, W_neigh, b, Wr0, br0, Wr1, br1):
  grid = (N // ROWS_BLK,)
  H1 = Wr0.shape[1]
  return pl.pallas_call(
      _tc_layer2_body,
      grid=grid,
      in_specs=[
          pl.BlockSpec((ROWS_BLK, D), lambda i: (i, 0)),
          pl.BlockSpec((NC, ROWS_BLK, D), lambda i: (0, i, 0)),
          pl.BlockSpec((NC, ROWS_BLK, 1), lambda i: (0, i, 0)),
          pl.BlockSpec((D, D), lambda i: (0, 0)),
          pl.BlockSpec((D, D), lambda i: (0, 0)),
          pl.BlockSpec((1, D), lambda i: (0, 0)),
          pl.BlockSpec((D, H1), lambda i: (0, 0)),
          pl.BlockSpec((1, H1), lambda i: (0, 0)),
          pl.BlockSpec((H1, 1), lambda i: (0, 0)),
          pl.BlockSpec((1, 1), lambda i: (0, 0)),
      ],
      out_specs=pl.BlockSpec((ROWS_BLK, 1), lambda i: (i, 0)),
      out_shape=jax.ShapeDtypeStruct((N, 1), jnp.float32),
  )(h, aggp, degs, W_self, W_neigh, b.reshape(1, D),
    Wr0, br0.reshape(1, H1), Wr1, br1.reshape(1, 1))


def kernel(x, edge_index, W_self0, W_neigh0, b0, W_self1, W_neigh1, b1,
           Wr0, br0, Wr1, br1):
  ei = edge_index.astype(jnp.int32)
  src = ei[0]
  dst = ei[1]
  zeros_rows = jnp.zeros((K, D), jnp.float32)
  ones_rows = jnp.ones((K, D), jnp.float32)

  degp = _sc_deg(dst, zeros_rows, ones_rows)
  degs = degp[:, :, :1]
  agg0 = _sc_agg(x, src, dst, zeros_rows)
  h1 = _tc_layer1(x, agg0, degs, W_self0, W_neigh0, b0)
  agg1 = _sc_agg(h1, src, dst, zeros_rows)
  return _tc_layer2(h1, agg1, degs, W_self1, W_neigh1, b1, Wr0, br0, Wr1, br1)
